# baseline (speedup 1.0000x reference)
import functools
import math

import jax
import jax.numpy as jnp
from jax.experimental import pallas as pl
from jax.experimental.pallas import tpu as pltpu

N_TOK = 4096
VOCAB = 32000
BT = 128
VC = 16000
NG = VC // 128
T_BLOCKS = N_TOK // BT
V_CHUNKS = VOCAB // VC

MIN_TOKENS_RATIO = 0.25
WARMUP_STEPS = 1000
THRESHOLD_SENSITIVITY = 0.5
STEP_COUNT = 0

_BIG_NEG = -3.4e38


def _stats_kernel(x_ref, tgt_ref, nll_ref, conf_ref, diff_ref,
                  m1_ref, m2_ref, s_ref, t_ref, tg_ref, gk_ref):
    j = pl.program_id(1)

    @pl.when(j == 0)
    def _init():
        m1_ref[...] = jnp.full((BT, 128), _BIG_NEG, jnp.float32)
        m2_ref[...] = jnp.full((BT, 128), _BIG_NEG, jnp.float32)
        s_ref[...] = jnp.zeros((BT, 128), jnp.float32)
        t_ref[...] = jnp.zeros((BT, 128), jnp.float32)
        tg_ref[...] = jnp.zeros((BT, 128), jnp.float32)
        lane = jax.lax.broadcasted_iota(jnp.int32, (BT, 128), 1)
        tgt = tgt_ref[0, 0, :]
        gk_ref[...] = jnp.where(lane == (tgt[:, None] & 127),
                                tgt[:, None] >> 7, -1)

    m1 = m1_ref[...]
    m2 = m2_ref[...]
    m1_old = m1
    for g in range(NG):
        v = x_ref[:, g * 128:(g + 1) * 128]
        m2 = jnp.maximum(m2, jnp.minimum(m1, v))
        m1 = jnp.maximum(m1, v)
    m1_ref[...] = m1
    m2_ref[...] = m2

    alpha = jnp.exp(m1_old - m1)
    s = s_ref[...] * alpha
    t = t_ref[...] * alpha
    tg = tg_ref[...]

    gk = gk_ref[...]
    g0 = j * NG
    for g in range(NG):
        v = x_ref[:, g * 128:(g + 1) * 128]
        e = jnp.exp(v - m1)
        s = s + e
        t = t + e * v
        tg = tg + jnp.where(gk == g0 + g, v, 0.0)
    s_ref[...] = s
    t_ref[...] = t
    tg_ref[...] = tg

    @pl.when(j == V_CHUNKS - 1)
    def _finalize():
        m1g = jnp.max(m1, axis=1, keepdims=True)
        ism = m1 == m1g
        cnt = jnp.sum(ism.astype(jnp.float32), axis=1, keepdims=True)
        sec1 = jnp.max(jnp.where(ism, _BIG_NEG, m1), axis=1, keepdims=True)
        sec1 = jnp.where(cnt > 1.0, m1g, sec1)
        m2g = jnp.maximum(jnp.max(m2, axis=1, keepdims=True), sec1)
        beta = jnp.exp(m1 - m1g)
        sg = jnp.sum(s * beta, axis=1, keepdims=True)
        tt = jnp.sum(t * beta, axis=1, keepdims=True)
        tgv = jnp.sum(tg, axis=1, keepdims=True)
        lse = m1g + jnp.log(sg)
        nll = lse - tgv
        entropy = lse - tt / sg
        confv = jnp.exp(m1g - lse)
        p2 = jnp.exp(m2g - lse)
        difficulty = entropy / math.log(float(VOCAB)) + (1.0 - (confv - p2)) + nll
        nll_ref[0, 0, :] = nll[:, 0]
        conf_ref[0, 0, :] = confv[:, 0]
        diff_ref[0, 0, :] = difficulty[:, 0]


def _mask_loss_kernel(d_row_ref, d_col_ref, nll_col_ref, conf_ref, out_ref):
    conf = conf_ref[...]
    avg_conf = jnp.sum(conf) / float(N_TOK)
    progress = min(1.0, STEP_COUNT / max(1, WARMUP_STEPS))
    base_ratio = 1.0 - progress * (1.0 - MIN_TOKENS_RATIO)
    ratio = jnp.clip(
        base_ratio * (1.0 + THRESHOLD_SENSITIVITY * (1.0 - 2.0 * avg_conf)),
        0.05, 1.0)
    k = jnp.maximum(1, jnp.floor(ratio * float(N_TOK)).astype(jnp.int32))
    kf = k.astype(jnp.float32)

    d_row = d_row_ref[...]
    RB = 512
    total = jnp.zeros((1, 1), jnp.float32)
    for b in range(N_TOK // RB):
        db = d_col_ref[pl.ds(b * RB, RB), :]
        nb = nll_col_ref[pl.ds(b * RB, RB), :]
        gt = (d_row > db).astype(jnp.float32)
        jj = jax.lax.broadcasted_iota(jnp.int32, (RB, N_TOK), 1)
        ii = jax.lax.broadcasted_iota(jnp.int32, (RB, N_TOK), 0) + b * RB
        tie = jnp.logical_and(d_row == db, jj < ii).astype(jnp.float32)
        rank = jnp.sum(gt + tie, axis=1, keepdims=True)
        sel = (rank < kf).astype(jnp.float32)
        total = total + jnp.sum(nb * sel, axis=0, keepdims=True)
    out_ref[...] = total / jnp.maximum(kf, 1.0)


@functools.partial(jax.jit, static_argnames=())
def kernel(logits, targets):
    lf = logits.reshape(N_TOK, VOCAB)
    tgt = targets.reshape(T_BLOCKS, 1, BT).astype(jnp.int32)

    nll, conf, diff = pl.pallas_call(
        _stats_kernel,
        grid=(T_BLOCKS, V_CHUNKS),
        in_specs=[
            pl.BlockSpec((BT, VC), lambda i, j: (i, j)),
            pl.BlockSpec((1, 1, BT), lambda i, j: (i, 0, 0)),
        ],
        out_specs=[
            pl.BlockSpec((1, 1, BT), lambda i, j: (i, 0, 0)),
            pl.BlockSpec((1, 1, BT), lambda i, j: (i, 0, 0)),
            pl.BlockSpec((1, 1, BT), lambda i, j: (i, 0, 0)),
        ],
        out_shape=[
            jax.ShapeDtypeStruct((T_BLOCKS, 1, BT), jnp.float32),
            jax.ShapeDtypeStruct((T_BLOCKS, 1, BT), jnp.float32),
            jax.ShapeDtypeStruct((T_BLOCKS, 1, BT), jnp.float32),
        ],
        scratch_shapes=[pltpu.VMEM((BT, 128), jnp.float32) for _ in range(5)]
        + [pltpu.VMEM((BT, 128), jnp.int32)],
        compiler_params=pltpu.CompilerParams(
            dimension_semantics=("parallel", "arbitrary")),
    )(lf, tgt)

    loss = pl.pallas_call(
        _mask_loss_kernel,
        out_shape=jax.ShapeDtypeStruct((1, 1), jnp.float32),
    )(diff.reshape(1, N_TOK), diff.reshape(N_TOK, 1), nll.reshape(N_TOK, 1),
      conf.reshape(1, N_TOK))
    return loss.reshape(())


# speedup vs baseline: 1.0353x; 1.0353x over previous
import functools
import math

import jax
import jax.numpy as jnp
from jax.experimental import pallas as pl
from jax.experimental.pallas import tpu as pltpu

N_TOK = 4096
VOCAB = 32000
BT = 128
VC = 32000
NG = VC // 128
T_BLOCKS = N_TOK // BT
V_CHUNKS = VOCAB // VC

MIN_TOKENS_RATIO = 0.25
WARMUP_STEPS = 1000
THRESHOLD_SENSITIVITY = 0.5
STEP_COUNT = 0

_BIG_NEG = -3.4e38


def _stats_kernel(x_ref, tgt_ref, nll_ref, conf_ref, diff_ref,
                  m1_ref, m2_ref, s_ref, t_ref, tg_ref):
    j = pl.program_id(1)

    @pl.when(j == 0)
    def _init():
        m1_ref[...] = jnp.full((BT, 128), _BIG_NEG, jnp.float32)
        m2_ref[...] = jnp.full((BT, 128), _BIG_NEG, jnp.float32)
        s_ref[...] = jnp.zeros((BT, 128), jnp.float32)
        t_ref[...] = jnp.zeros((BT, 128), jnp.float32)
        tg_ref[...] = jnp.zeros((BT, 128), jnp.float32)

    m1 = m1_ref[...]
    m2 = m2_ref[...]
    m1_old = m1
    for g in range(NG):
        v = x_ref[:, g * 128:(g + 1) * 128]
        m2 = jnp.maximum(m2, jnp.minimum(m1, v))
        m1 = jnp.maximum(m1, v)
    m1_ref[...] = m1
    m2_ref[...] = m2

    alpha = jnp.exp(m1_old - m1)
    s = s_ref[...] * alpha
    t = t_ref[...] * alpha
    tg = tg_ref[...]

    lane = jax.lax.broadcasted_iota(jnp.int32, (BT, 128), 1)
    tgt = tgt_ref[0, 0, :]
    rel = tgt[:, None] - j * VC
    for g in range(NG):
        v = x_ref[:, g * 128:(g + 1) * 128]
        e = jnp.exp(v - m1)
        s = s + e
        t = t + e * v
        hit = lane == (rel - g * 128)
        tg = tg + jnp.where(hit, v, 0.0)
    s_ref[...] = s
    t_ref[...] = t
    tg_ref[...] = tg

    @pl.when(j == V_CHUNKS - 1)
    def _finalize():
        m1g = jnp.max(m1, axis=1, keepdims=True)
        ism = m1 == m1g
        cnt = jnp.sum(ism.astype(jnp.float32), axis=1, keepdims=True)
        sec1 = jnp.max(jnp.where(ism, _BIG_NEG, m1), axis=1, keepdims=True)
        sec1 = jnp.where(cnt > 1.0, m1g, sec1)
        m2g = jnp.maximum(jnp.max(m2, axis=1, keepdims=True), sec1)
        beta = jnp.exp(m1 - m1g)
        sg = jnp.sum(s * beta, axis=1, keepdims=True)
        tt = jnp.sum(t * beta, axis=1, keepdims=True)
        tgv = jnp.sum(tg, axis=1, keepdims=True)
        lse = m1g + jnp.log(sg)
        nll = lse - tgv
        entropy = lse - tt / sg
        confv = jnp.exp(m1g - lse)
        p2 = jnp.exp(m2g - lse)
        difficulty = entropy / math.log(float(VOCAB)) + (1.0 - (confv - p2)) + nll
        nll_ref[0, 0, :] = nll[:, 0]
        conf_ref[0, 0, :] = confv[:, 0]
        diff_ref[0, 0, :] = difficulty[:, 0]


def _mask_loss_kernel(d_row_ref, d_col_ref, nll_col_ref, conf_ref, out_ref):
    conf = conf_ref[...]
    avg_conf = jnp.sum(conf) / float(N_TOK)
    progress = min(1.0, STEP_COUNT / max(1, WARMUP_STEPS))
    base_ratio = 1.0 - progress * (1.0 - MIN_TOKENS_RATIO)
    ratio = jnp.clip(
        base_ratio * (1.0 + THRESHOLD_SENSITIVITY * (1.0 - 2.0 * avg_conf)),
        0.05, 1.0)
    k = jnp.maximum(1, jnp.floor(ratio * float(N_TOK)).astype(jnp.int32))
    kf = k.astype(jnp.float32)

    d_row = d_row_ref[...]
    RB = 512
    total = jnp.zeros((1, 1), jnp.float32)
    for b in range(N_TOK // RB):
        db = d_col_ref[pl.ds(b * RB, RB), :]
        nb = nll_col_ref[pl.ds(b * RB, RB), :]
        gt = (d_row > db).astype(jnp.float32)
        jj = jax.lax.broadcasted_iota(jnp.int32, (RB, N_TOK), 1)
        ii = jax.lax.broadcasted_iota(jnp.int32, (RB, N_TOK), 0) + b * RB
        tie = jnp.logical_and(d_row == db, jj < ii).astype(jnp.float32)
        rank = jnp.sum(gt + tie, axis=1, keepdims=True)
        sel = (rank < kf).astype(jnp.float32)
        total = total + jnp.sum(nb * sel, axis=0, keepdims=True)
    out_ref[...] = total / jnp.maximum(kf, 1.0)


@functools.partial(jax.jit, static_argnames=())
def kernel(logits, targets):
    lf = logits.reshape(N_TOK, VOCAB)
    tgt = targets.reshape(T_BLOCKS, 1, BT).astype(jnp.int32)

    nll, conf, diff = pl.pallas_call(
        _stats_kernel,
        grid=(T_BLOCKS, V_CHUNKS),
        in_specs=[
            pl.BlockSpec((BT, VC), lambda i, j: (i, j)),
            pl.BlockSpec((1, 1, BT), lambda i, j: (i, 0, 0)),
        ],
        out_specs=[
            pl.BlockSpec((1, 1, BT), lambda i, j: (i, 0, 0)),
            pl.BlockSpec((1, 1, BT), lambda i, j: (i, 0, 0)),
            pl.BlockSpec((1, 1, BT), lambda i, j: (i, 0, 0)),
        ],
        out_shape=[
            jax.ShapeDtypeStruct((T_BLOCKS, 1, BT), jnp.float32),
            jax.ShapeDtypeStruct((T_BLOCKS, 1, BT), jnp.float32),
            jax.ShapeDtypeStruct((T_BLOCKS, 1, BT), jnp.float32),
        ],
        scratch_shapes=[pltpu.VMEM((BT, 128), jnp.float32) for _ in range(5)],
        compiler_params=pltpu.CompilerParams(
            dimension_semantics=("parallel", "arbitrary")),
    )(lf, tgt)

    loss = pl.pallas_call(
        _mask_loss_kernel,
        out_shape=jax.ShapeDtypeStruct((1, 1), jnp.float32),
    )(diff.reshape(1, N_TOK), diff.reshape(N_TOK, 1), nll.reshape(N_TOK, 1),
      conf.reshape(1, N_TOK))
    return loss.reshape(())


# speedup vs baseline: 1.0825x; 1.0457x over previous
import functools
import math

import jax
import jax.numpy as jnp
from jax.experimental import pallas as pl
from jax.experimental.pallas import tpu as pltpu

N_TOK = 4096
VOCAB = 32000
BT = 128
VC = 16000
NG = VC // 128
T_BLOCKS = N_TOK // BT
V_CHUNKS = VOCAB // VC

MIN_TOKENS_RATIO = 0.25
WARMUP_STEPS = 1000
THRESHOLD_SENSITIVITY = 0.5
STEP_COUNT = 0

_BIG_NEG = -3.4e38


def _stats_kernel(x_ref, tgt_ref, nll_ref, conf_ref, diff_ref,
                  m1_ref, m2_ref, s_ref, t_ref, tg_ref):
    j = pl.program_id(1)

    @pl.when(j == 0)
    def _init():
        m1_ref[...] = jnp.full((BT, 128), _BIG_NEG, jnp.float32)
        m2_ref[...] = jnp.full((BT, 128), _BIG_NEG, jnp.float32)
        s_ref[...] = jnp.zeros((BT, 128), jnp.float32)
        t_ref[...] = jnp.zeros((BT, 128), jnp.float32)
        tg_ref[...] = jnp.zeros((BT, 128), jnp.float32)

    m1 = m1_ref[...]
    m2 = m2_ref[...]
    m1_old = m1
    for g in range(NG):
        v = x_ref[:, g * 128:(g + 1) * 128]
        m2 = jnp.maximum(m2, jnp.minimum(m1, v))
        m1 = jnp.maximum(m1, v)
    m1_ref[...] = m1
    m2_ref[...] = m2

    alpha = jnp.exp(m1_old - m1)
    s = s_ref[...] * alpha
    t = t_ref[...] * alpha
    tg = tg_ref[...]

    lane = jax.lax.broadcasted_iota(jnp.int32, (BT, 128), 1)
    tgt = tgt_ref[0, 0, :]
    rel = tgt[:, None] - j * VC
    for g in range(NG):
        v = x_ref[:, g * 128:(g + 1) * 128]
        e = jnp.exp(v - m1)
        s = s + e
        t = t + e * v
        hit = lane == (rel - g * 128)
        tg = jnp.where(hit, v, tg)
    s_ref[...] = s
    t_ref[...] = t
    tg_ref[...] = tg

    @pl.when(j == V_CHUNKS - 1)
    def _finalize():
        m1g = jnp.max(m1, axis=1, keepdims=True)
        ism = m1 == m1g
        cnt = jnp.sum(ism.astype(jnp.float32), axis=1, keepdims=True)
        sec1 = jnp.max(jnp.where(ism, _BIG_NEG, m1), axis=1, keepdims=True)
        sec1 = jnp.where(cnt > 1.0, m1g, sec1)
        m2g = jnp.maximum(jnp.max(m2, axis=1, keepdims=True), sec1)
        beta = jnp.exp(m1 - m1g)
        sg = jnp.sum(s * beta, axis=1, keepdims=True)
        tt = jnp.sum(t * beta, axis=1, keepdims=True)
        tgv = jnp.sum(tg, axis=1, keepdims=True)
        lse = m1g + jnp.log(sg)
        nll = lse - tgv
        entropy = lse - tt / sg
        confv = jnp.exp(m1g - lse)
        p2 = jnp.exp(m2g - lse)
        difficulty = entropy / math.log(float(VOCAB)) + (1.0 - (confv - p2)) + nll
        nll_ref[0, 0, :] = nll[:, 0]
        conf_ref[0, 0, :] = confv[:, 0]
        diff_ref[0, 0, :] = difficulty[:, 0]


def _mask_loss_kernel(d_row_ref, d_col_ref, nll_col_ref, conf_ref, out_ref):
    conf = conf_ref[...]
    avg_conf = jnp.sum(conf) / float(N_TOK)
    progress = min(1.0, STEP_COUNT / max(1, WARMUP_STEPS))
    base_ratio = 1.0 - progress * (1.0 - MIN_TOKENS_RATIO)
    ratio = jnp.clip(
        base_ratio * (1.0 + THRESHOLD_SENSITIVITY * (1.0 - 2.0 * avg_conf)),
        0.05, 1.0)
    k = jnp.maximum(1, jnp.floor(ratio * float(N_TOK)).astype(jnp.int32))
    kf = k.astype(jnp.float32)

    d_row = d_row_ref[...]
    RB = 512
    total = jnp.zeros((1, 1), jnp.float32)
    for b in range(N_TOK // RB):
        db = d_col_ref[pl.ds(b * RB, RB), :]
        nb = nll_col_ref[pl.ds(b * RB, RB), :]
        gt = (d_row > db).astype(jnp.float32)
        jj = jax.lax.broadcasted_iota(jnp.int32, (RB, N_TOK), 1)
        ii = jax.lax.broadcasted_iota(jnp.int32, (RB, N_TOK), 0) + b * RB
        tie = jnp.logical_and(d_row == db, jj < ii).astype(jnp.float32)
        rank = jnp.sum(gt + tie, axis=1, keepdims=True)
        sel = (rank < kf).astype(jnp.float32)
        total = total + jnp.sum(nb * sel, axis=0, keepdims=True)
    out_ref[...] = total / jnp.maximum(kf, 1.0)


@functools.partial(jax.jit, static_argnames=())
def kernel(logits, targets):
    lf = logits.reshape(N_TOK, VOCAB)
    tgt = targets.reshape(T_BLOCKS, 1, BT).astype(jnp.int32)

    nll, conf, diff = pl.pallas_call(
        _stats_kernel,
        grid=(T_BLOCKS, V_CHUNKS),
        in_specs=[
            pl.BlockSpec((BT, VC), lambda i, j: (i, j)),
            pl.BlockSpec((1, 1, BT), lambda i, j: (i, 0, 0)),
        ],
        out_specs=[
            pl.BlockSpec((1, 1, BT), lambda i, j: (i, 0, 0)),
            pl.BlockSpec((1, 1, BT), lambda i, j: (i, 0, 0)),
            pl.BlockSpec((1, 1, BT), lambda i, j: (i, 0, 0)),
        ],
        out_shape=[
            jax.ShapeDtypeStruct((T_BLOCKS, 1, BT), jnp.float32),
            jax.ShapeDtypeStruct((T_BLOCKS, 1, BT), jnp.float32),
            jax.ShapeDtypeStruct((T_BLOCKS, 1, BT), jnp.float32),
        ],
        scratch_shapes=[pltpu.VMEM((BT, 128), jnp.float32) for _ in range(5)],
        compiler_params=pltpu.CompilerParams(
            dimension_semantics=("parallel", "arbitrary")),
    )(lf, tgt)

    loss = pl.pallas_call(
        _mask_loss_kernel,
        out_shape=jax.ShapeDtypeStruct((1, 1), jnp.float32),
    )(diff.reshape(1, N_TOK), diff.reshape(N_TOK, 1), nll.reshape(N_TOK, 1),
      conf.reshape(1, N_TOK))
    return loss.reshape(())


# speedup vs baseline: 1.0989x; 1.0151x over previous
import functools
import math

import jax
import jax.numpy as jnp
from jax.experimental import pallas as pl
from jax.experimental.pallas import tpu as pltpu

N_TOK = 4096
VOCAB = 32000
BT = 128
VC = 16000
NG = VC // 128
T_BLOCKS = N_TOK // BT
V_CHUNKS = VOCAB // VC

MIN_TOKENS_RATIO = 0.25
WARMUP_STEPS = 1000
THRESHOLD_SENSITIVITY = 0.5
STEP_COUNT = 0

_BIG_NEG = -3.4e38


def _stats_kernel(x_ref, tgt_ref, nll_ref, conf_ref, diff_ref,
                  m1_ref, m2_ref, s_ref, t_ref, tg_ref):
    j = pl.program_id(1)

    @pl.when(j == 0)
    def _init():
        m1_ref[...] = jnp.full((BT, 128), _BIG_NEG, jnp.float32)
        m2_ref[...] = jnp.full((BT, 128), _BIG_NEG, jnp.float32)
        s_ref[...] = jnp.zeros((BT, 128), jnp.float32)
        t_ref[...] = jnp.zeros((BT, 128), jnp.float32)
        tg_ref[...] = jnp.zeros((BT, 128), jnp.float32)

    m1 = m1_ref[...]
    m2 = m2_ref[...]
    m1_old = m1
    for g in range(NG):
        v = x_ref[:, g * 128:(g + 1) * 128]
        m2 = jnp.maximum(m2, jnp.minimum(m1, v))
        m1 = jnp.maximum(m1, v)
    m1_ref[...] = m1
    m2_ref[...] = m2

    alpha = jnp.exp(m1_old - m1)
    s = s_ref[...] * alpha
    t = t_ref[...] * alpha
    tg = tg_ref[...]

    lane = jax.lax.broadcasted_iota(jnp.int32, (BT, 128), 1)
    tgt = tgt_ref[0, 0, :]
    rel = tgt[:, None] - j * VC
    s1 = jnp.zeros((BT, 128), jnp.float32)
    t1 = jnp.zeros((BT, 128), jnp.float32)
    for g in range(NG):
        v = x_ref[:, g * 128:(g + 1) * 128]
        e = jnp.exp(v - m1)
        if g % 2 == 0:
            s = s + e
            t = t + e * v
        else:
            s1 = s1 + e
            t1 = t1 + e * v
        hit = lane == (rel - g * 128)
        tg = tg + jnp.where(hit, v, 0.0)
    s_ref[...] = s + s1
    t_ref[...] = t + t1
    tg_ref[...] = tg

    @pl.when(j == V_CHUNKS - 1)
    def _finalize():
        m1g = jnp.max(m1, axis=1, keepdims=True)
        ism = m1 == m1g
        cnt = jnp.sum(ism.astype(jnp.float32), axis=1, keepdims=True)
        sec1 = jnp.max(jnp.where(ism, _BIG_NEG, m1), axis=1, keepdims=True)
        sec1 = jnp.where(cnt > 1.0, m1g, sec1)
        m2g = jnp.maximum(jnp.max(m2, axis=1, keepdims=True), sec1)
        beta = jnp.exp(m1 - m1g)
        sg = jnp.sum(s * beta, axis=1, keepdims=True)
        tt = jnp.sum(t * beta, axis=1, keepdims=True)
        tgv = jnp.sum(tg, axis=1, keepdims=True)
        lse = m1g + jnp.log(sg)
        nll = lse - tgv
        entropy = lse - tt / sg
        confv = jnp.exp(m1g - lse)
        p2 = jnp.exp(m2g - lse)
        difficulty = entropy / math.log(float(VOCAB)) + (1.0 - (confv - p2)) + nll
        nll_ref[0, 0, :] = nll[:, 0]
        conf_ref[0, 0, :] = confv[:, 0]
        diff_ref[0, 0, :] = difficulty[:, 0]


def _mask_loss_kernel(d_row_ref, d_col_ref, nll_col_ref, conf_ref, out_ref):
    conf = conf_ref[...]
    avg_conf = jnp.sum(conf) / float(N_TOK)
    progress = min(1.0, STEP_COUNT / max(1, WARMUP_STEPS))
    base_ratio = 1.0 - progress * (1.0 - MIN_TOKENS_RATIO)
    ratio = jnp.clip(
        base_ratio * (1.0 + THRESHOLD_SENSITIVITY * (1.0 - 2.0 * avg_conf)),
        0.05, 1.0)
    k = jnp.maximum(1, jnp.floor(ratio * float(N_TOK)).astype(jnp.int32))
    kf = k.astype(jnp.float32)

    d_row = d_row_ref[...]
    RB = 512
    total = jnp.zeros((1, 1), jnp.float32)
    for b in range(N_TOK // RB):
        db = d_col_ref[pl.ds(b * RB, RB), :]
        nb = nll_col_ref[pl.ds(b * RB, RB), :]
        gt = (d_row > db).astype(jnp.float32)
        jj = jax.lax.broadcasted_iota(jnp.int32, (RB, N_TOK), 1)
        ii = jax.lax.broadcasted_iota(jnp.int32, (RB, N_TOK), 0) + b * RB
        tie = jnp.logical_and(d_row == db, jj < ii).astype(jnp.float32)
        rank = jnp.sum(gt + tie, axis=1, keepdims=True)
        sel = (rank < kf).astype(jnp.float32)
        total = total + jnp.sum(nb * sel, axis=0, keepdims=True)
    out_ref[...] = total / jnp.maximum(kf, 1.0)


@functools.partial(jax.jit, static_argnames=())
def kernel(logits, targets):
    lf = logits.reshape(N_TOK, VOCAB)
    tgt = targets.reshape(T_BLOCKS, 1, BT).astype(jnp.int32)

    nll, conf, diff = pl.pallas_call(
        _stats_kernel,
        grid=(T_BLOCKS, V_CHUNKS),
        in_specs=[
            pl.BlockSpec((BT, VC), lambda i, j: (i, j)),
            pl.BlockSpec((1, 1, BT), lambda i, j: (i, 0, 0)),
        ],
        out_specs=[
            pl.BlockSpec((1, 1, BT), lambda i, j: (i, 0, 0)),
            pl.BlockSpec((1, 1, BT), lambda i, j: (i, 0, 0)),
            pl.BlockSpec((1, 1, BT), lambda i, j: (i, 0, 0)),
        ],
        out_shape=[
            jax.ShapeDtypeStruct((T_BLOCKS, 1, BT), jnp.float32),
            jax.ShapeDtypeStruct((T_BLOCKS, 1, BT), jnp.float32),
            jax.ShapeDtypeStruct((T_BLOCKS, 1, BT), jnp.float32),
        ],
        scratch_shapes=[pltpu.VMEM((BT, 128), jnp.float32) for _ in range(5)],
        compiler_params=pltpu.CompilerParams(
            dimension_semantics=("parallel", "arbitrary")),
    )(lf, tgt)

    loss = pl.pallas_call(
        _mask_loss_kernel,
        out_shape=jax.ShapeDtypeStruct((1, 1), jnp.float32),
    )(diff.reshape(1, N_TOK), diff.reshape(N_TOK, 1), nll.reshape(N_TOK, 1),
      conf.reshape(1, N_TOK))
    return loss.reshape(())
